# baseline (speedup 1.0000x reference)
import functools

import jax
import jax.numpy as jnp
from jax import lax
from jax.experimental import pallas as pl
from jax.experimental.pallas import tpu as pltpu
from jax.experimental.pallas import tpu_sc as plsc

_NC = 2
_NS = 16
_L = 16

_CHUNK = 128


@functools.lru_cache(maxsize=None)
def _make_sc_spmm(N, D, R, EPTP, NPAD):
    assert R % _NC == 0 and NPAD % (8 * _NS) == 0 and EPTP % (2 * _CHUNK) == 0
    rpc = R // _NC
    n_chunks = EPTP // _CHUNK
    n_pairs = n_chunks // 2
    rpt = NPAD // _NS
    mesh = plsc.VectorSubcoreMesh(core_axis_name="c", subcore_axis_name="s")

    @functools.partial(
        pl.kernel,
        out_type=jax.ShapeDtypeStruct((R * NPAD, D), jnp.float32),
        mesh=mesh,
        scratch_types=[
            pltpu.VMEM((2, _CHUNK), jnp.int32),
            pltpu.VMEM((2, _CHUNK), jnp.int32),
            pltpu.VMEM((_CHUNK,), jnp.float32),
            pltpu.VMEM((_CHUNK,), jnp.float32),
            pltpu.VMEM((_CHUNK, D), jnp.float32),
            pltpu.VMEM((_CHUNK, D), jnp.float32),
            pltpu.VMEM_SHARED((NPAD, D), jnp.float32),
            pltpu.SemaphoreType.DMA,
            pltpu.SemaphoreType.DMA,
        ],
    )
    def spmm(x_hbm, packed_hbm, w_hbm, out_hbm,
             idx0, idx1, w0, w1, rows0, rows1, acc, sem0, sem1):
        cid = lax.axis_index("c")
        sid = lax.axis_index("s")
        row0 = sid * rpt
        idx = (idx0, idx1)
        wv = (w0, w1)
        rows = (rows0, rows1)
        sem = (sem0, sem1)

        def _scale(rows_b, w_b):
            def body(c16, c2):
                wvec = w_b[pl.ds(c16 * _L, _L)]
                for e16 in range(_L):
                    w = wvec[e16]
                    e = c16 * _L + e16
                    for j in range(D // _L):
                        sl = pl.ds(j * _L, _L)
                        rows_b[e, sl] = rows_b[e, sl] * w
                return c2
            lax.fori_loop(0, _CHUNK // _L, body, 0)

        def _half(seg, k, b):
            pltpu.make_async_copy(x_hbm.at[idx[b].at[0]], rows[b],
                                  sem[b]).wait()
            _scale(rows[b], wv[b])
            pltpu.sync_copy(rows[b], acc.at[idx[b].at[1]], add=True)
            pltpu.sync_copy(packed_hbm.at[seg + k + 2], idx[b])
            pltpu.sync_copy(w_hbm.at[seg + k + 2], wv[b])
            nb = 1 - b
            pltpu.async_copy(x_hbm.at[idx[nb].at[0]], rows[nb], sem[nb])

        for rr in range(rpc):
            r = cid * rpc + rr
            seg = (r * _NS + sid) * n_chunks

            def _zb(i, carry):
                z = jnp.zeros((_L,), jnp.float32)
                for j in range(D // _L):
                    rows1[i, pl.ds(j * _L, _L)] = z
                return carry
            lax.fori_loop(0, _CHUNK, _zb, 0)
            done = 0
            while done < rpt:
                nrows = min(_CHUNK, rpt - done)
                pltpu.sync_copy(rows1.at[pl.ds(0, nrows)],
                                acc.at[pl.ds(row0 + done, nrows)])
                done += nrows

            pltpu.sync_copy(packed_hbm.at[seg + 0], idx0)
            pltpu.sync_copy(w_hbm.at[seg + 0], w0)
            pltpu.sync_copy(packed_hbm.at[seg + 1], idx1)
            pltpu.sync_copy(w_hbm.at[seg + 1], w1)
            pltpu.async_copy(x_hbm.at[idx0.at[0]], rows0, sem0)
            plsc.subcore_barrier()

            def _pair(k2, carry):
                k = k2 * 2
                _half(seg, k, 0)
                _half(seg, k + 1, 1)
                return carry
            lax.fori_loop(0, n_pairs - 1, _pair, 0)

            pltpu.async_copy(x_hbm.at[idx1.at[0]], rows1, sem1)
            pltpu.make_async_copy(x_hbm.at[idx0.at[0]], rows0, sem0).wait()
            _scale(rows0, w0)
            pltpu.sync_copy(rows0, acc.at[idx0.at[1]], add=True)
            pltpu.make_async_copy(x_hbm.at[idx1.at[0]], rows1, sem1).wait()
            _scale(rows1, w1)
            pltpu.sync_copy(rows1, acc.at[idx1.at[1]], add=True)
            plsc.subcore_barrier()

            done = 0
            while done < rpt:
                nrows = min(_CHUNK, rpt - done)
                pltpu.sync_copy(acc.at[pl.ds(row0 + done, nrows)],
                                out_hbm.at[pl.ds(r * NPAD + row0 + done,
                                                 nrows)])
                done += nrows

    return spmm


@functools.lru_cache(maxsize=None)
def _make_dense(N, D, R, DOUT, B):
    assert N % B == 0
    grid = (N // B,)

    def body(s_ref, w_ref, sw_ref, b_ref, g_ref, be_ref, o_ref):
        ssum = s_ref[0]
        for r in range(1, R):
            ssum = ssum + s_ref[r]
        mu = jnp.mean(ssum, axis=-1, keepdims=True)
        d = ssum - mu
        var = jnp.mean(d * d, axis=-1, keepdims=True)
        sn = d * lax.rsqrt(var + 1e-6) * g_ref[...] + be_ref[...]
        acc = jnp.dot(sn, sw_ref[...], preferred_element_type=jnp.float32)
        for r in range(R):
            acc = acc + jnp.dot(s_ref[r], w_ref[r],
                                preferred_element_type=jnp.float32)
        o_ref[...] = acc * 0.5 + b_ref[...]

    return pl.pallas_call(
        body,
        grid=grid,
        in_specs=[
            pl.BlockSpec((R, B, D), lambda i: (0, i, 0)),
            pl.BlockSpec((R, D, DOUT), lambda i: (0, 0, 0)),
            pl.BlockSpec((D, DOUT), lambda i: (0, 0)),
            pl.BlockSpec((1, DOUT), lambda i: (0, 0)),
            pl.BlockSpec((1, D), lambda i: (0, 0)),
            pl.BlockSpec((1, D), lambda i: (0, 0)),
        ],
        out_specs=pl.BlockSpec((B, DOUT), lambda i: (i, 0)),
        out_shape=jax.ShapeDtypeStruct((N, DOUT), jnp.float32),
    )


def kernel(x, edge_index, edge_weight, weight, share_weight, bias,
           ln_gamma, ln_beta):
    N, D = x.shape
    R, _, E = edge_index.shape
    DOUT = weight.shape[1]

    ept = E // _NS
    n_chunks = -(-ept // _CHUNK)
    n_chunks += n_chunks % 2
    eptp = n_chunks * _CHUNK
    pad = eptp - ept

    src = edge_index[:, 0, :].reshape(R, _NS, ept)
    dst = edge_index[:, 1, :].reshape(R, _NS, ept)
    ew = edge_weight.reshape(R, _NS, ept)
    if pad:
        src = jnp.pad(src, ((0, 0), (0, 0), (0, pad)))
        dst = jnp.pad(dst, ((0, 0), (0, 0), (0, pad)))
        ew = jnp.pad(ew, ((0, 0), (0, 0), (0, pad)))
    packed = jnp.stack(
        [src.reshape(R * _NS, n_chunks, _CHUNK),
         dst.reshape(R * _NS, n_chunks, _CHUNK)], axis=2)
    packed = packed.reshape(R * _NS * n_chunks, 2, _CHUNK)
    wchunks = ew.reshape(R * _NS * n_chunks, _CHUNK)

    npad = -(-N // (8 * _NS)) * (8 * _NS)
    supports = _make_sc_spmm(N, D, R, eptp, npad)(x, packed, wchunks)
    s = supports.reshape(R, npad, D)

    dense = _make_dense(N, D, R, DOUT, B=1000)
    return dense(
        s,
        weight.reshape(R, D, DOUT),
        share_weight,
        bias.reshape(1, DOUT),
        ln_gamma.reshape(1, D),
        ln_beta.reshape(1, D),
    )


# speedup vs baseline: 1.2238x; 1.2238x over previous
import functools

import jax
import jax.numpy as jnp
from jax import lax
from jax.experimental import pallas as pl
from jax.experimental.pallas import tpu as pltpu
from jax.experimental.pallas import tpu_sc as plsc

_NC = 2
_NS = 16
_L = 16

_CHUNK = 128


@functools.lru_cache(maxsize=None)
def _make_sc_spmm(N, D, R, EPTP, NPAD):
    assert R % _NC == 0 and NPAD % (8 * _NS) == 0 and EPTP % (4 * _CHUNK) == 0
    rpc = R // _NC
    n_chunks = EPTP // _CHUNK
    assert n_chunks >= 8
    rpt = NPAD // _NS
    mesh = plsc.VectorSubcoreMesh(core_axis_name="c", subcore_axis_name="s")

    @functools.partial(
        pl.kernel,
        out_type=jax.ShapeDtypeStruct((R * NPAD, D), jnp.float32),
        mesh=mesh,
        scratch_types=[
            [pltpu.VMEM((2, _CHUNK), jnp.int32) for _ in range(4)],
            [pltpu.VMEM((_CHUNK,), jnp.float32) for _ in range(4)],
            [pltpu.VMEM((_CHUNK, D), jnp.float32) for _ in range(2)],
            pltpu.VMEM_SHARED((NPAD, D), jnp.float32),
            [pltpu.SemaphoreType.DMA for _ in range(2)],
            [pltpu.SemaphoreType.DMA for _ in range(2)],
            [pltpu.SemaphoreType.DMA for _ in range(4)],
        ],
    )
    def spmm(x_hbm, packed_hbm, w_hbm, out_hbm,
             idx, wv, rows, acc, sem_g, sem_s, sem_i):
        cid = lax.axis_index("c")
        sid = lax.axis_index("s")
        row0 = sid * rpt

        def _scale(rows_b, w_b):
            def body(c16, c2):
                wvec = w_b[pl.ds(c16 * _L, _L)]
                for e16 in range(_L):
                    w = wvec[e16]
                    e = c16 * _L + e16
                    for j in range(D // _L):
                        sl = pl.ds(j * _L, _L)
                        rows_b[e, sl] = rows_b[e, sl] * w
                return c2
            lax.fori_loop(0, _CHUNK // _L, body, 0)

        def _stage_idx(seg, k, q):
            pltpu.async_copy(packed_hbm.at[seg + k], idx[q], sem_i[q])
            pltpu.async_copy(w_hbm.at[seg + k], wv[q], sem_i[q])

        def _wait_idx(seg, k, q):
            pltpu.make_async_copy(packed_hbm.at[seg + k], idx[q],
                                  sem_i[q]).wait()
            pltpu.make_async_copy(w_hbm.at[seg + k], wv[q], sem_i[q]).wait()

        def _start_gather(b, q):
            pltpu.async_copy(x_hbm.at[idx[q].at[0]], rows[b], sem_g[b])

        def _wait_gather(b, q):
            pltpu.make_async_copy(x_hbm.at[idx[q].at[0]], rows[b],
                                  sem_g[b]).wait()

        def _start_scatter(b, q):
            pltpu.async_copy(rows[b], acc.at[idx[q].at[1]], sem_s[b],
                             add=True)

        def _wait_scatter(b, q):
            pltpu.make_async_copy(rows[b], acc.at[idx[q].at[1]],
                                  sem_s[b]).wait()

        def _half(seg, k, b, q, first, second, lastpre, last):
            nb = 1 - b
            q1 = (q + 1) % 4
            q2 = (q + 2) % 4
            _wait_gather(b, q)
            _scale(rows[b], wv[q])
            _start_scatter(b, q)
            if not (lastpre or last):
                _stage_idx(seg, k + 2, q2)
            if not (first or last):
                _wait_scatter(nb, (q + 3) % 4)
            if not last:
                if not first:
                    _wait_idx(seg, k + 1, q1)
                _start_gather(nb, q1)
            if last:
                _wait_scatter(nb, (q + 3) % 4)
                _wait_scatter(b, q)

        for rr in range(rpc):
            r = cid * rpc + rr
            seg = (r * _NS + sid) * n_chunks

            def _zb(i, carry):
                z = jnp.zeros((_L,), jnp.float32)
                for j in range(D // _L):
                    rows[1][i, pl.ds(j * _L, _L)] = z
                return carry
            lax.fori_loop(0, _CHUNK, _zb, 0)
            done = 0
            while done < rpt:
                nrows = min(_CHUNK, rpt - done)
                pltpu.sync_copy(rows[1].at[pl.ds(0, nrows)],
                                acc.at[pl.ds(row0 + done, nrows)])
                done += nrows

            pltpu.sync_copy(packed_hbm.at[seg + 0], idx[0])
            pltpu.sync_copy(w_hbm.at[seg + 0], wv[0])
            pltpu.sync_copy(packed_hbm.at[seg + 1], idx[1])
            pltpu.sync_copy(w_hbm.at[seg + 1], wv[1])
            _start_gather(0, 0)
            plsc.subcore_barrier()

            _half(seg, 0, 0, 0, True, False, False, False)
            _half(seg, 1, 1, 1, False, True, False, False)

            def _quad(i, carry):
                k = 2 + i * 4
                _half(seg, k + 0, 0, 2, False, False, False, False)
                _half(seg, k + 1, 1, 3, False, False, False, False)
                _half(seg, k + 2, 0, 0, False, False, False, False)
                _half(seg, k + 3, 1, 1, False, False, False, False)
                return carry
            lax.fori_loop(0, (n_chunks - 4) // 4, _quad, 0)

            _half(seg, n_chunks - 2, 0, (n_chunks - 2) % 4,
                  False, False, True, False)
            _half(seg, n_chunks - 1, 1, (n_chunks - 1) % 4,
                  False, False, False, True)
            plsc.subcore_barrier()

            done = 0
            while done < rpt:
                nrows = min(_CHUNK, rpt - done)
                pltpu.sync_copy(acc.at[pl.ds(row0 + done, nrows)],
                                out_hbm.at[pl.ds(r * NPAD + row0 + done,
                                                 nrows)])
                done += nrows

    return spmm


@functools.lru_cache(maxsize=None)
def _make_dense(N, D, R, DOUT, B):
    assert N % B == 0
    grid = (N // B,)

    def body(s_ref, w_ref, sw_ref, b_ref, g_ref, be_ref, o_ref):
        ssum = s_ref[0]
        for r in range(1, R):
            ssum = ssum + s_ref[r]
        mu = jnp.mean(ssum, axis=-1, keepdims=True)
        d = ssum - mu
        var = jnp.mean(d * d, axis=-1, keepdims=True)
        sn = d * lax.rsqrt(var + 1e-6) * g_ref[...] + be_ref[...]
        acc = jnp.dot(sn, sw_ref[...], preferred_element_type=jnp.float32)
        for r in range(R):
            acc = acc + jnp.dot(s_ref[r], w_ref[r],
                                preferred_element_type=jnp.float32)
        o_ref[...] = acc * 0.5 + b_ref[...]

    return pl.pallas_call(
        body,
        grid=grid,
        in_specs=[
            pl.BlockSpec((R, B, D), lambda i: (0, i, 0)),
            pl.BlockSpec((R, D, DOUT), lambda i: (0, 0, 0)),
            pl.BlockSpec((D, DOUT), lambda i: (0, 0)),
            pl.BlockSpec((1, DOUT), lambda i: (0, 0)),
            pl.BlockSpec((1, D), lambda i: (0, 0)),
            pl.BlockSpec((1, D), lambda i: (0, 0)),
        ],
        out_specs=pl.BlockSpec((B, DOUT), lambda i: (i, 0)),
        out_shape=jax.ShapeDtypeStruct((N, DOUT), jnp.float32),
    )


def kernel(x, edge_index, edge_weight, weight, share_weight, bias,
           ln_gamma, ln_beta):
    N, D = x.shape
    R, _, E = edge_index.shape
    DOUT = weight.shape[1]

    ept = E // _NS
    n_chunks = -(-ept // _CHUNK)
    n_chunks = -(-n_chunks // 4) * 4
    eptp = n_chunks * _CHUNK
    pad = eptp - ept

    src = edge_index[:, 0, :].reshape(R, _NS, ept)
    dst = edge_index[:, 1, :].reshape(R, _NS, ept)
    ew = edge_weight.reshape(R, _NS, ept)
    if pad:
        src = jnp.pad(src, ((0, 0), (0, 0), (0, pad)))
        dst = jnp.pad(dst, ((0, 0), (0, 0), (0, pad)))
        ew = jnp.pad(ew, ((0, 0), (0, 0), (0, pad)))
    packed = jnp.stack(
        [src.reshape(R * _NS, n_chunks, _CHUNK),
         dst.reshape(R * _NS, n_chunks, _CHUNK)], axis=2)
    packed = packed.reshape(R * _NS * n_chunks, 2, _CHUNK)
    wchunks = ew.reshape(R * _NS * n_chunks, _CHUNK)

    npad = -(-N // (8 * _NS)) * (8 * _NS)
    supports = _make_sc_spmm(N, D, R, eptp, npad)(x, packed, wchunks)
    s = supports.reshape(R, npad, D)

    dense = _make_dense(N, D, R, DOUT, B=1000)
    return dense(
        s,
        weight.reshape(R, D, DOUT),
        share_weight,
        bias.reshape(1, DOUT),
        ln_gamma.reshape(1, D),
        ln_beta.reshape(1, D),
    )


# speedup vs baseline: 1.2676x; 1.0357x over previous
import functools

import jax
import jax.numpy as jnp
from jax import lax
from jax.experimental import pallas as pl
from jax.experimental.pallas import tpu as pltpu
from jax.experimental.pallas import tpu_sc as plsc

_NC = 2
_NS = 16
_L = 16

_CHUNK = 128
_GROUP = 8


@functools.lru_cache(maxsize=None)
def _make_sc_spmm(N, D, R, EPTP, NPAD):
    assert (R % _NC == 0 and NPAD % (8 * _NS) == 0
            and EPTP % (_GROUP * _CHUNK) == 0)
    rpc = R // _NC
    n_chunks = EPTP // _CHUNK
    n_groups = n_chunks // _GROUP
    rpt = NPAD // _NS
    mesh = plsc.VectorSubcoreMesh(core_axis_name="c", subcore_axis_name="s")

    @functools.partial(
        pl.kernel,
        out_type=jax.ShapeDtypeStruct((R * NPAD, D), jnp.float32),
        mesh=mesh,
        scratch_types=[
            pltpu.VMEM((_GROUP, 2, _CHUNK), jnp.int32),
            pltpu.VMEM((_GROUP, _CHUNK), jnp.float32),
            [pltpu.VMEM((_CHUNK, D), jnp.float32) for _ in range(2)],
            pltpu.VMEM_SHARED((NPAD, D), jnp.float32),
            [pltpu.SemaphoreType.DMA for _ in range(2)],
        ],
    )
    def spmm(x_hbm, packed_hbm, w_hbm, out_hbm, idx_g, w_g, rows, acc, sem):
        cid = lax.axis_index("c")
        sid = lax.axis_index("s")
        row0 = sid * rpt

        def _scale(rows_b, kk):
            def body(c16, c2):
                wvec = w_g[kk, pl.ds(c16 * _L, _L)]
                for e16 in range(_L):
                    w = wvec[e16]
                    e = c16 * _L + e16
                    for j in range(D // _L):
                        sl = pl.ds(j * _L, _L)
                        rows_b[e, sl] = rows_b[e, sl] * w
                return c2
            lax.fori_loop(0, _CHUNK // _L, body, 0)

        for rr in range(rpc):
            r = cid * rpc + rr

            def _zb(i, carry):
                z = jnp.zeros((_L,), jnp.float32)
                for j in range(D // _L):
                    rows[1][i, pl.ds(j * _L, _L)] = z
                return carry
            lax.fori_loop(0, _CHUNK, _zb, 0)
            done = 0
            while done < rpt:
                nrows = min(_CHUNK, rpt - done)
                pltpu.sync_copy(rows[1].at[pl.ds(0, nrows)],
                                acc.at[pl.ds(row0 + done, nrows)])
                done += nrows
            plsc.subcore_barrier()

            gseg = (r * _NS + sid) * n_groups

            def _group(g, carry):
                pltpu.sync_copy(packed_hbm.at[gseg + g], idx_g)
                pltpu.sync_copy(w_hbm.at[gseg + g], w_g)
                descs = [None] * _GROUP
                for kk in range(2):
                    descs[kk] = pltpu.async_copy(
                        x_hbm.at[idx_g.at[kk, 0]], rows[kk % 2], sem[kk % 2])
                for kk in range(_GROUP):
                    b = kk % 2
                    descs[kk].wait()
                    _scale(rows[b], kk)
                    pltpu.sync_copy(rows[b], acc.at[idx_g.at[kk, 1]],
                                    add=True)
                    if kk + 2 < _GROUP:
                        descs[kk + 2] = pltpu.async_copy(
                            x_hbm.at[idx_g.at[kk + 2, 0]], rows[b], sem[b])
                return carry
            lax.fori_loop(0, n_groups, _group, 0)
            plsc.subcore_barrier()

            done = 0
            while done < rpt:
                nrows = min(_CHUNK, rpt - done)
                pltpu.sync_copy(acc.at[pl.ds(row0 + done, nrows)],
                                out_hbm.at[pl.ds(r * NPAD + row0 + done,
                                                 nrows)])
                done += nrows

    return spmm


@functools.lru_cache(maxsize=None)
def _make_dense(N, D, R, DOUT, B):
    assert N % B == 0
    grid = (N // B,)

    def body(s_ref, w_ref, sw_ref, b_ref, g_ref, be_ref, o_ref):
        ssum = s_ref[0]
        for r in range(1, R):
            ssum = ssum + s_ref[r]
        mu = jnp.mean(ssum, axis=-1, keepdims=True)
        d = ssum - mu
        var = jnp.mean(d * d, axis=-1, keepdims=True)
        sn = d * lax.rsqrt(var + 1e-6) * g_ref[...] + be_ref[...]
        acc = jnp.dot(sn, sw_ref[...], preferred_element_type=jnp.float32)
        for r in range(R):
            acc = acc + jnp.dot(s_ref[r], w_ref[r],
                                preferred_element_type=jnp.float32)
        o_ref[...] = acc * 0.5 + b_ref[...]

    return pl.pallas_call(
        body,
        grid=grid,
        in_specs=[
            pl.BlockSpec((R, B, D), lambda i: (0, i, 0)),
            pl.BlockSpec((R, D, DOUT), lambda i: (0, 0, 0)),
            pl.BlockSpec((D, DOUT), lambda i: (0, 0)),
            pl.BlockSpec((1, DOUT), lambda i: (0, 0)),
            pl.BlockSpec((1, D), lambda i: (0, 0)),
            pl.BlockSpec((1, D), lambda i: (0, 0)),
        ],
        out_specs=pl.BlockSpec((B, DOUT), lambda i: (i, 0)),
        out_shape=jax.ShapeDtypeStruct((N, DOUT), jnp.float32),
    )


def kernel(x, edge_index, edge_weight, weight, share_weight, bias,
           ln_gamma, ln_beta):
    N, D = x.shape
    R, _, E = edge_index.shape
    DOUT = weight.shape[1]

    ept = E // _NS
    n_chunks = -(-ept // _CHUNK)
    n_chunks = -(-n_chunks // _GROUP) * _GROUP
    eptp = n_chunks * _CHUNK
    pad = eptp - ept

    src = edge_index[:, 0, :].reshape(R, _NS, ept)
    dst = edge_index[:, 1, :].reshape(R, _NS, ept)
    ew = edge_weight.reshape(R, _NS, ept)
    if pad:
        src = jnp.pad(src, ((0, 0), (0, 0), (0, pad)))
        dst = jnp.pad(dst, ((0, 0), (0, 0), (0, pad)))
        ew = jnp.pad(ew, ((0, 0), (0, 0), (0, pad)))
    n_groups = n_chunks // _GROUP
    packed = jnp.stack(
        [src.reshape(R * _NS, n_chunks, _CHUNK),
         dst.reshape(R * _NS, n_chunks, _CHUNK)], axis=2)
    packed = packed.reshape(R * _NS * n_groups, _GROUP, 2, _CHUNK)
    wchunks = ew.reshape(R * _NS * n_groups, _GROUP, _CHUNK)

    npad = -(-N // (8 * _NS)) * (8 * _NS)
    supports = _make_sc_spmm(N, D, R, eptp, npad)(x, packed, wchunks)
    s = supports.reshape(R, npad, D)

    dense = _make_dense(N, D, R, DOUT, B=1000)
    return dense(
        s,
        weight.reshape(R, D, DOUT),
        share_weight,
        bias.reshape(1, DOUT),
        ln_gamma.reshape(1, D),
        ln_beta.reshape(1, D),
    )
